# Initial kernel scaffold; baseline (speedup 1.0000x reference)
import jax
import jax.numpy as jnp
from jax.experimental import pallas as pl


def kernel(user_id, U, V):
    raise NotImplementedError("write your pallas kernel here")


# speedup vs baseline: 1.1439x; 1.1439x over previous
import jax
import jax.numpy as jnp
from jax.experimental import pallas as pl
from jax.experimental.pallas import tpu as pltpu

_N_ITEMS = 1_000_000
_RANK = 32
_BLOCK = 65536
_GRID = (_N_ITEMS + _BLOCK - 1) // _BLOCK


def _score_body(uid_ref, ub_ref, vt_ref, out_ref):
    c = uid_ref[0] % 128
    lane = jax.lax.broadcasted_iota(jnp.int32, (_RANK, 128), 1)
    u_col = jnp.sum(
        jnp.where(lane == c, ub_ref[...], 0.0), axis=1, keepdims=True
    )
    scores = jax.lax.dot_general(
        u_col,
        vt_ref[...],
        dimension_numbers=(((0,), (0,)), ((), ())),
        preferred_element_type=jnp.float32,
    )
    out_ref[...] = scores.reshape((_BLOCK,))


def kernel(user_id, U, V):
    uid = jnp.asarray(user_id, jnp.int32).reshape((1,))
    ut = U.T
    vt = V.T
    grid_spec = pltpu.PrefetchScalarGridSpec(
        num_scalar_prefetch=1,
        grid=(_GRID,),
        in_specs=[
            pl.BlockSpec((_RANK, 128), lambda i, uid_ref: (0, uid_ref[0] // 128)),
            pl.BlockSpec((_RANK, _BLOCK), lambda i, uid_ref: (0, i)),
        ],
        out_specs=pl.BlockSpec((_BLOCK,), lambda i, uid_ref: (i,)),
    )
    return pl.pallas_call(
        _score_body,
        grid_spec=grid_spec,
        out_shape=jax.ShapeDtypeStruct((_N_ITEMS,), jnp.float32),
    )(uid, ut, vt)
